# baseline (speedup 1.0000x reference)
import jax
import jax.numpy as jnp
from jax import lax
from jax.experimental import pallas as pl
from jax.experimental.pallas import tpu as pltpu
from jax.experimental.pallas import tpu_sc as plsc

NX, NY, NZ = 128, 128, 256
N = NX * NY
NNZ = 268435

NC = 2
NS = 16
CW = 32
NCHUNK = NZ // CW
CHUNKS_PER_SC = NCHUNK // NC
BATCH = 128
NB = -(-NNZ // (NS * BATCH))
NNZ_PAD = NS * NB * BATCH
ROWS_PER_TILE = N // NS
K = 4


def _mm_body(x_ref, w_ref, o_ref):
    o_ref[...] = jnp.dot(x_ref[...], w_ref[...],
                         preferred_element_type=jnp.float32)


def _matmul(squ, mat_z):
    BM = 1024
    return pl.pallas_call(
        _mm_body,
        grid=(N // BM,),
        in_specs=[
            pl.BlockSpec((BM, NZ), lambda i: (i, 0)),
            pl.BlockSpec((NZ, NZ), lambda i: (0, 0)),
        ],
        out_specs=pl.BlockSpec((BM, NZ), lambda i: (i, 0)),
        out_shape=jax.ShapeDtypeStruct((N, NZ), jnp.float32),
    )(squ, mat_z)


def _sc_body(table, packed_h, vals_h, out,
             idx_v, cols_v, vals_v,
             data0, data1, data2, data3, acc_ref,
             gs0, gs1, gs2, gs3, ss0, ss1, ss2, ss3):
    c = lax.axis_index("c")
    s = lax.axis_index("s")
    data = (data0, data1, data2, data3)
    gsem = (gs0, gs1, gs2, gs3)
    ssem = (ss0, ss1, ss2, ss3)

    pltpu.sync_copy(packed_h.at[s], idx_v)
    pltpu.sync_copy(vals_h.at[s], vals_v)

    maskv = jnp.full((16,), N - 1, jnp.int32)
    zvec = jnp.zeros((16,), jnp.float32)

    def _unpack_body(b, _):
        for j in range(BATCH // 16):
            w = idx_v[b, pl.ds(j * 16, 16)]
            cols_v[b, pl.ds(j * 16, 16)] = w & maskv
            idx_v[b, pl.ds(j * 16, 16)] = (
                lax.shift_right_logical(w, jnp.full((16,), 11, jnp.int32))
                & jnp.full((16,), ~7, jnp.int32))
        return 0

    lax.fori_loop(0, NB, _unpack_body, 0)

    for i in range(CHUNKS_PER_SC):
        chunk = CHUNKS_PER_SC * c + i
        delta = chunk if i == 0 else 1
        deltav = lax.broadcast(jnp.int32(delta), (16,))

        def _shift_body(b, _):
            for j in range(BATCH // 16):
                idx_v[b, pl.ds(j * 16, 16)] = (
                    idx_v[b, pl.ds(j * 16, 16)] + deltav)
            return 0

        lax.fori_loop(0, NB, _shift_body, 0)

        def _zero_body(r, _):
            for k in range(CW // 16):
                data0[r, pl.ds(k * 16, 16)] = zvec
            return 0

        lax.fori_loop(0, BATCH, _zero_body, 0)
        for z in range(ROWS_PER_TILE // BATCH):
            pltpu.sync_copy(
                data0, acc_ref.at[pl.ds(s * ROWS_PER_TILE + z * BATCH, BATCH)])
        plsc.subcore_barrier()

        for k in range(2):
            pltpu.async_copy(table.at[idx_v.at[k]], data[k], gsem[k])

        iotav = lax.iota(jnp.int32, 16)

        def _scale(buf, b):
            def _rows(j, _):
                rowv = iotav + lax.broadcast(j * 16, (16,))
                vv = vals_v[b, pl.ds(j * 16, 16)]
                for k in range(CW):
                    colv = jnp.full((16,), k, jnp.int32)
                    g = plsc.load_gather(buf, [rowv, colv])
                    plsc.store_scatter(buf, [rowv, colv], g * vv)
                return 0
            lax.fori_loop(0, BATCH // 16, _rows, 0)

        def _ring_body(g, _):
            for k in range(K):
                b = g * K + k
                pltpu.make_async_copy(table.at[idx_v.at[k]],
                                      data[k], gsem[k]).wait()
                _scale(data[k], b)
                pltpu.async_copy(data[k], acc_ref.at[cols_v.at[b]],
                                 ssem[k], add=True)
                kp = (k + 2) % K
                bp = b + 2

                @pl.when(bp < NB)
                def _():
                    @pl.when(bp - K >= 0)
                    def _():
                        pltpu.make_async_copy(
                            data[kp], acc_ref.at[cols_v.at[bp - K]],
                            ssem[kp]).wait()
                    pltpu.async_copy(table.at[idx_v.at[bp]], data[kp],
                                     gsem[kp])
            return 0

        lax.fori_loop(0, NB // K, _ring_body, 0)
        for k in range(K):
            b = NB - K + k
            pltpu.make_async_copy(data[k], acc_ref.at[cols_v.at[b]],
                                  ssem[k]).wait()
        plsc.subcore_barrier()

        pltpu.sync_copy(
            acc_ref.at[pl.ds(s * ROWS_PER_TILE, ROWS_PER_TILE)],
            out.at[pl.ds(s * ROWS_PER_TILE, ROWS_PER_TILE),
                   pl.ds(chunk * CW, CW)])
        plsc.subcore_barrier()


def _sc_scatter(table, packed_r, vals_r):
    mesh = plsc.VectorSubcoreMesh(core_axis_name="c", subcore_axis_name="s")
    f = pl.kernel(
        _sc_body,
        mesh=mesh,
        compiler_params=pltpu.CompilerParams(use_tc_tiling_on_sc=False,
                                             needs_layout_passes=False),
        out_type=jax.ShapeDtypeStruct((N, NZ), jnp.float32),
        scratch_types=[
            pltpu.VMEM((NB, BATCH), jnp.int32),
            pltpu.VMEM((NB, BATCH), jnp.int32),
            pltpu.VMEM((NB, BATCH), jnp.float32),
            pltpu.VMEM((BATCH, CW), jnp.float32),
            pltpu.VMEM((BATCH, CW), jnp.float32),
            pltpu.VMEM((BATCH, CW), jnp.float32),
            pltpu.VMEM((BATCH, CW), jnp.float32),
            pltpu.VMEM_SHARED((N, CW), jnp.float32),
            pltpu.SemaphoreType.DMA,
            pltpu.SemaphoreType.DMA,
            pltpu.SemaphoreType.DMA,
            pltpu.SemaphoreType.DMA,
            pltpu.SemaphoreType.DMA,
            pltpu.SemaphoreType.DMA,
            pltpu.SemaphoreType.DMA,
            pltpu.SemaphoreType.DMA,
        ],
    )
    return f(table, packed_r, vals_r)


def kernel(bproj, mat_xy_indices, mat_xy_values, mat_z):
    squ = bproj.reshape(N, NZ)
    result1 = _matmul(squ, mat_z)
    table = result1.reshape(N * NCHUNK, CW)

    rows = mat_xy_indices[:, 0]
    cols = mat_xy_indices[:, 1]
    packed = rows * N + cols
    pad = NNZ_PAD - NNZ
    pad_idx = (jnp.arange(pad, dtype=jnp.int32) * 37) % N
    packed_p = jnp.concatenate(
        [packed, pad_idx * N + pad_idx]).reshape(NS, NB, BATCH)
    vals_p = jnp.concatenate(
        [mat_xy_values, jnp.zeros((pad,), jnp.float32)]).reshape(NS, NB, BATCH)

    out = _sc_scatter(table, packed_p, vals_p)
    return out.reshape(NX, NY, NZ)


# speedup vs baseline: 7.1566x; 7.1566x over previous
import jax
import jax.numpy as jnp
from jax import lax
from jax.experimental import pallas as pl
from jax.experimental.pallas import tpu as pltpu
from jax.experimental.pallas import tpu_sc as plsc

NX, NY, NZ = 128, 128, 256
N = NX * NY
NNZ = 268435

NC = 2
NS = 16
CW = 32
NCHUNK = NZ // CW
CHUNKS_PER_SC = NCHUNK // NC
BATCH = 128
NB = -(-NNZ // (NS * BATCH))
NNZ_PAD = NS * NB * BATCH
ROWS_PER_TILE = N // NS
K = 4


def _mm_body(x_ref, w_ref, o_ref):
    o_ref[...] = jnp.dot(x_ref[...], w_ref[...],
                         preferred_element_type=jnp.float32)


def _matmul(squ, mat_z):
    BM = 1024
    return pl.pallas_call(
        _mm_body,
        grid=(N // BM,),
        in_specs=[
            pl.BlockSpec((BM, NZ), lambda i: (i, 0)),
            pl.BlockSpec((NZ, NZ), lambda i: (0, 0)),
        ],
        out_specs=pl.BlockSpec((BM, NZ), lambda i: (i, 0)),
        out_shape=jax.ShapeDtypeStruct((N, NZ), jnp.float32),
    )(squ, mat_z)


def _sc_body(table, packed_h, vals_h, out,
             idx_v, cols_v, vals_v,
             data0, data1, data2, data3, acc_ref,
             gs0, gs1, gs2, gs3, ss0, ss1, ss2, ss3):
    c = lax.axis_index("c")
    s = lax.axis_index("s")
    data = (data0, data1, data2, data3)
    gsem = (gs0, gs1, gs2, gs3)
    ssem = (ss0, ss1, ss2, ss3)

    pltpu.sync_copy(packed_h.at[s], idx_v)
    pltpu.sync_copy(vals_h.at[s], vals_v)

    maskv = jnp.full((16,), N - 1, jnp.int32)
    zvec = jnp.zeros((16,), jnp.float32)

    def _unpack_body(b, _):
        for j in range(BATCH // 16):
            w = idx_v[b, pl.ds(j * 16, 16)]
            cols_v[b, pl.ds(j * 16, 16)] = w & maskv
            idx_v[b, pl.ds(j * 16, 16)] = (
                lax.shift_right_logical(w, jnp.full((16,), 11, jnp.int32))
                & jnp.full((16,), ~7, jnp.int32))
        return 0

    lax.fori_loop(0, NB, _unpack_body, 0)

    for i in range(CHUNKS_PER_SC):
        chunk = CHUNKS_PER_SC * c + i
        delta = chunk if i == 0 else 1
        deltav = lax.broadcast(jnp.int32(delta), (16,))

        def _shift_body(b, _):
            for j in range(BATCH // 16):
                idx_v[b, pl.ds(j * 16, 16)] = (
                    idx_v[b, pl.ds(j * 16, 16)] + deltav)
            return 0

        lax.fori_loop(0, NB, _shift_body, 0)

        def _zero_body(r, _):
            for k in range(CW // 16):
                data0[r, pl.ds(k * 16, 16)] = zvec
            return 0

        lax.fori_loop(0, BATCH, _zero_body, 0)
        for z in range(ROWS_PER_TILE // BATCH):
            pltpu.sync_copy(
                data0, acc_ref.at[pl.ds(s * ROWS_PER_TILE + z * BATCH, BATCH)])
        plsc.subcore_barrier()

        for k in range(2):
            pltpu.async_copy(table.at[idx_v.at[k]], data[k], gsem[k])

        def _scale(buf, b):
            def _rows(j, _):
                vv = vals_v[b, pl.ds(j * 16, 16)]
                for l in range(16):
                    vsp = lax.broadcast(vv[l], (16,))
                    r = j * 16 + l
                    for k in range(CW // 16):
                        buf[r, pl.ds(k * 16, 16)] = (
                            buf[r, pl.ds(k * 16, 16)] * vsp)
                return 0
            lax.fori_loop(0, BATCH // 16, _rows, 0)

        def _ring_body(g, _):
            for k in range(K):
                b = g * K + k
                pltpu.make_async_copy(table.at[idx_v.at[k]],
                                      data[k], gsem[k]).wait()
                _scale(data[k], b)
                pltpu.async_copy(data[k], acc_ref.at[cols_v.at[b]],
                                 ssem[k], add=True)
                kp = (k + 2) % K
                bp = b + 2

                @pl.when(bp < NB)
                def _():
                    @pl.when(bp - K >= 0)
                    def _():
                        pltpu.make_async_copy(
                            data[kp], acc_ref.at[cols_v.at[bp - K]],
                            ssem[kp]).wait()
                    pltpu.async_copy(table.at[idx_v.at[bp]], data[kp],
                                     gsem[kp])
            return 0

        lax.fori_loop(0, NB // K, _ring_body, 0)
        for k in range(K):
            b = NB - K + k
            pltpu.make_async_copy(data[k], acc_ref.at[cols_v.at[b]],
                                  ssem[k]).wait()
        plsc.subcore_barrier()

        pltpu.sync_copy(
            acc_ref.at[pl.ds(s * ROWS_PER_TILE, ROWS_PER_TILE)],
            out.at[pl.ds(s * ROWS_PER_TILE, ROWS_PER_TILE),
                   pl.ds(chunk * CW, CW)])
        plsc.subcore_barrier()


def _sc_scatter(table, packed_r, vals_r):
    mesh = plsc.VectorSubcoreMesh(core_axis_name="c", subcore_axis_name="s")
    f = pl.kernel(
        _sc_body,
        mesh=mesh,
        compiler_params=pltpu.CompilerParams(use_tc_tiling_on_sc=False,
                                             needs_layout_passes=False),
        out_type=jax.ShapeDtypeStruct((N, NZ), jnp.float32),
        scratch_types=[
            pltpu.VMEM((NB, BATCH), jnp.int32),
            pltpu.VMEM((NB, BATCH), jnp.int32),
            pltpu.VMEM((NB, BATCH), jnp.float32),
            pltpu.VMEM((BATCH, CW), jnp.float32),
            pltpu.VMEM((BATCH, CW), jnp.float32),
            pltpu.VMEM((BATCH, CW), jnp.float32),
            pltpu.VMEM((BATCH, CW), jnp.float32),
            pltpu.VMEM_SHARED((N, CW), jnp.float32),
            pltpu.SemaphoreType.DMA,
            pltpu.SemaphoreType.DMA,
            pltpu.SemaphoreType.DMA,
            pltpu.SemaphoreType.DMA,
            pltpu.SemaphoreType.DMA,
            pltpu.SemaphoreType.DMA,
            pltpu.SemaphoreType.DMA,
            pltpu.SemaphoreType.DMA,
        ],
    )
    return f(table, packed_r, vals_r)


def kernel(bproj, mat_xy_indices, mat_xy_values, mat_z):
    squ = bproj.reshape(N, NZ)
    result1 = _matmul(squ, mat_z)
    table = result1.reshape(N * NCHUNK, CW)

    rows = mat_xy_indices[:, 0]
    cols = mat_xy_indices[:, 1]
    packed = rows * N + cols
    pad = NNZ_PAD - NNZ
    pad_idx = (jnp.arange(pad, dtype=jnp.int32) * 37) % N
    packed_p = jnp.concatenate(
        [packed, pad_idx * N + pad_idx]).reshape(NS, NB, BATCH)
    vals_p = jnp.concatenate(
        [mat_xy_values, jnp.zeros((pad,), jnp.float32)]).reshape(NS, NB, BATCH)

    out = _sc_scatter(table, packed_p, vals_p)
    return out.reshape(NX, NY, NZ)
